# baseline (speedup 1.0000x reference)
import jax
import jax.numpy as jnp
from jax.experimental import pallas as pl


def _rs_kernel(pp_ref, w_ref, b_ref, out_ref):
    x = pp_ref[:, 0, :, :]
    n = x.shape[1]
    m = jnp.sum(x, axis=1) * (1.0 / n)
    logits = jax.lax.dot_general(
        m, w_ref[...], (((1,), (1,)), ((), ())),
        preferred_element_type=jnp.float32,
    )
    out_ref[...] = logits + b_ref[...]


def kernel(cls_embedding, predicted_path, sep_index_list, W, b, root):
    Bb, _, N, H = predicted_path.shape
    C = W.shape[0]
    b2 = b.reshape(1, C)
    return pl.pallas_call(
        _rs_kernel,
        grid=(1,),
        in_specs=[
            pl.BlockSpec((Bb, 1, N, H), lambda i: (0, 0, 0, 0)),
            pl.BlockSpec((C, H), lambda i: (0, 0)),
            pl.BlockSpec((1, C), lambda i: (0, 0)),
        ],
        out_specs=pl.BlockSpec((Bb, C), lambda i: (0, 0)),
        out_shape=jax.ShapeDtypeStruct((Bb, C), jnp.float32),
    )(predicted_path, W, b2)


# speedup vs baseline: 11.6814x; 11.6814x over previous
import jax
import jax.numpy as jnp
from jax.experimental import pallas as pl


def _rs_kernel(pp_ref, w_ref, b_ref, out_ref):
    x = pp_ref[...]
    n = x.shape[1]
    m = jnp.sum(x, axis=1) * (1.0 / n)
    logits = jax.lax.dot_general(
        m, w_ref[...], (((1,), (1,)), ((), ())),
        preferred_element_type=jnp.float32,
    )
    out_ref[...] = logits + b_ref[...]


def kernel(cls_embedding, predicted_path, sep_index_list, W, b, root):
    Bb, _, N, H = predicted_path.shape
    C = W.shape[0]
    b2 = b.reshape(1, C)
    pp0 = predicted_path[:, 0]
    return pl.pallas_call(
        _rs_kernel,
        in_specs=[
            pl.BlockSpec((Bb, N, H), lambda: (0, 0, 0)),
            pl.BlockSpec((C, H), lambda: (0, 0)),
            pl.BlockSpec((1, C), lambda: (0, 0)),
        ],
        out_specs=pl.BlockSpec((Bb, C), lambda: (0, 0)),
        out_shape=jax.ShapeDtypeStruct((Bb, C), jnp.float32),
    )(pp0, W, b2)
